# Initial kernel scaffold; baseline (speedup 1.0000x reference)
import jax
import jax.numpy as jnp
from jax.experimental import pallas as pl


def kernel(pos1, batch1, pos2, batch2, params_sa1, params_sa2, params_sa3, lin_params):
    raise NotImplementedError("write your pallas kernel here")


# speedup vs baseline: 6.6426x; 6.6426x over previous
import functools

import numpy as np
import jax
import jax.numpy as jnp
from jax.experimental import pallas as pl

B, N = 8, 1024
M1, M2 = 512, 128
R1, R2 = 0.2, 0.4
KNBR = 64
EPS = 1e-5
NEG = np.float32(-np.inf)


def _fps_kernel(p_ref, s_ref, *, m, n, nb):
    p = p_ref[...]
    px, py, pz = p[0], p[1], p[2]
    iota = jax.lax.broadcasted_iota(jnp.int32, (nb, n), 1)

    def body(i, carry):
        dist, fx, fy, fz = carry
        s_ref[pl.ds(i, 1)] = jnp.concatenate([fx, fy, fz], axis=1)[None]
        d = (px - fx) ** 2 + (py - fy) ** 2 + (pz - fz) ** 2
        dist = jnp.minimum(dist, d)
        mx = jnp.max(dist, axis=1, keepdims=True)
        far = jnp.min(jnp.where(dist == mx, iota, n), axis=1, keepdims=True)
        sel = iota == far
        fx = jnp.sum(jnp.where(sel, px, 0.0), axis=1, keepdims=True)
        fy = jnp.sum(jnp.where(sel, py, 0.0), axis=1, keepdims=True)
        fz = jnp.sum(jnp.where(sel, pz, 0.0), axis=1, keepdims=True)
        return dist, fx, fy, fz

    dist0 = jnp.full((nb, n), jnp.inf, jnp.float32)
    f0 = (px[:, 0:1], py[:, 0:1], pz[:, 0:1])
    jax.lax.fori_loop(0, m, body, (dist0,) + f0)


def _fps_call(pt, m, interpret=False):
    _, nb, n = pt.shape
    return pl.pallas_call(
        functools.partial(_fps_kernel, m=m, n=n, nb=nb),
        out_shape=jax.ShapeDtypeStruct((m, nb, 3), jnp.float32),
        interpret=interpret,
    )(pt)


def _sel_kernel(p_ref, s_ref, t_ref, jc_ref, *, n, ti, rr, rr_bits, kk):
    p = p_ref[0]
    s = s_ref[0]
    d2 = ((s[:, 0:1] - p[0][None, :]) ** 2
          + (s[:, 1:2] - p[1][None, :]) ** 2
          + (s[:, 2:3] - p[2][None, :]) ** 2)
    bits = jax.lax.bitcast_convert_type(d2, jnp.int32)
    inr = d2 <= rr
    iota = jax.lax.broadcasted_iota(jnp.int32, (ti, n), 1)
    counts = jnp.sum(inr.astype(jnp.int32), axis=1, keepdims=True)

    def fast():
        return (jnp.full((ti, 1), rr_bits, jnp.int32),
                jnp.full((ti, 1), n, jnp.int32))

    def slow():
        lo = jnp.zeros((ti, 1), jnp.int32)
        hi = jnp.full((ti, 1), rr_bits, jnp.int32)

        def bs(_, lh):
            lo, hi = lh
            mid = lo + (hi - lo) // 2
            c = jnp.sum((inr & (bits <= mid)).astype(jnp.int32),
                        axis=1, keepdims=True)
            ge = c >= kk
            return jnp.where(ge, lo, mid + 1), jnp.where(ge, mid, hi)

        lo, hi = jax.lax.fori_loop(0, 31, bs, (lo, hi))
        t = hi
        cnt_lt = jnp.sum((inr & (bits < t)).astype(jnp.int32),
                         axis=1, keepdims=True)
        need = kk - cnt_lt
        lo2 = jnp.zeros((ti, 1), jnp.int32)
        hi2 = jnp.full((ti, 1), n - 1, jnp.int32)

        def bs2(_, lh):
            lo2, hi2 = lh
            mid = lo2 + (hi2 - lo2) // 2
            c = jnp.sum((inr & (bits == t) & (iota <= mid)).astype(jnp.int32),
                        axis=1, keepdims=True)
            ge = c >= need
            return jnp.where(ge, lo2, mid + 1), jnp.where(ge, mid, hi2)

        lo2, hi2 = jax.lax.fori_loop(0, 11, bs2, (lo2, hi2))
        over = counts > kk
        return (jnp.where(over, t, rr_bits),
                jnp.where(over, hi2, n))

    t, jc = jax.lax.cond(jnp.max(counts) > kk, slow, fast)
    t_ref[0] = t
    jc_ref[0] = jc


def _sel_call(pt, s, rr, ti, kk, interpret=False):
    nb, _, n = pt.shape
    m = s.shape[1]
    it = m // ti
    rr_bits = int(np.float32(rr).view(np.int32))
    grid = (nb, it)
    out_sh = jax.ShapeDtypeStruct((nb * it, ti, 1), jnp.int32)
    return pl.pallas_call(
        functools.partial(_sel_kernel, n=n, ti=ti, rr=np.float32(rr),
                          rr_bits=rr_bits, kk=kk),
        grid=grid,
        in_specs=[
            pl.BlockSpec((1, 3, n), lambda b, i: (b, 0, 0)),
            pl.BlockSpec((1, ti, 3), lambda b, i: (b, i, 0)),
        ],
        out_specs=[
            pl.BlockSpec((1, ti, 1), lambda b, i, it=it: (b * it + i, 0, 0)),
            pl.BlockSpec((1, ti, 1), lambda b, i, it=it: (b * it + i, 0, 0)),
        ],
        out_shape=[out_sh, out_sh],
        interpret=interpret,
    )(pt, s)


def _agg_kernel(*refs, njc, ti, jsz, rr, has_x):
    if has_x:
        (p_ref, s_ref, x_ref, t_ref, jc_ref, w1p_ref, w1x_ref,
         b1_ref, a1_ref, e1_ref, w2_ref, b2_ref, a2_ref, e2_ref,
         w3_ref, b3_ref, a3_ref, e3_ref, out_ref) = refs
    else:
        (p_ref, s_ref, t_ref, jc_ref, w1p_ref,
         b1_ref, a1_ref, e1_ref, w2_ref, b2_ref, a2_ref, e2_ref,
         w3_ref, b3_ref, a3_ref, e3_ref, out_ref) = refs
        x_ref = w1x_ref = None
    jcid = pl.program_id(2)

    p = p_ref[0]
    s = s_ref[0]
    d2 = ((s[:, 0:1] - p[0][None, :]) ** 2
          + (s[:, 1:2] - p[1][None, :]) ** 2
          + (s[:, 2:3] - p[2][None, :]) ** 2)
    bits = jax.lax.bitcast_convert_type(d2, jnp.int32)
    t = t_ref[0]
    jc = jc_ref[0]
    jg = jax.lax.broadcasted_iota(jnp.int32, (ti, jsz), 1) + jcid * jsz
    sel = (d2 <= rr) & ((bits < t) | ((bits == t) & (jg <= jc)))

    w1p = w1p_ref[...]
    dj = jax.lax.dot_general(p, w1p, (((0,), (0,)), ((), ())),
                             preferred_element_type=jnp.float32)
    if has_x:
        dj = dj + jnp.dot(x_ref[0], w1x_ref[...],
                          preferred_element_type=jnp.float32)
    ci = jnp.dot(s, w1p, preferred_element_type=jnp.float32)

    c1 = w1p.shape[-1]
    bc3 = lambda v, dims, sh: jax.lax.broadcast_in_dim(v, sh, dims)
    sh1 = (ti, jsz, c1)
    h = (bc3(dj, (1, 2), sh1) - bc3(ci, (0, 2), sh1)
         + bc3(b1_ref[...], (1, 2), sh1))
    h = jnp.maximum(h, 0.0) * bc3(a1_ref[...], (1, 2), sh1) \
        + bc3(e1_ref[...], (1, 2), sh1)
    h = h.reshape(ti * jsz, c1)
    h = jnp.dot(h, w2_ref[...], preferred_element_type=jnp.float32) + b2_ref[...]
    h = jnp.maximum(h, 0.0) * a2_ref[...] + e2_ref[...]
    h = jnp.dot(h, w3_ref[...], preferred_element_type=jnp.float32) + b3_ref[...]
    h = jnp.maximum(h, 0.0) * a3_ref[...] + e3_ref[...]
    c3 = h.shape[-1]
    h = h.reshape(ti, jsz, c3)
    pen = jnp.where(sel, 0.0, NEG)
    cm = jnp.max(h + jax.lax.broadcast_in_dim(pen, (ti, jsz, c3), (0, 1)),
                 axis=1)

    @pl.when(jcid == 0)
    def _():
        out_ref[0] = jnp.full((ti, c3), NEG, jnp.float32)

    out_ref[0] = jnp.maximum(out_ref[0], cm)

    @pl.when(jcid == njc - 1)
    def _():
        v = out_ref[0]
        out_ref[0] = jnp.where(v == NEG, 0.0, v)


def _agg_call(pt, s, x, tb, jc, lw, rr, ti, jsz, interpret=False):
    nb, _, n = pt.shape
    m = s.shape[1]
    it = m // ti
    njc = n // jsz
    has_x = x is not None
    (w1, b1, a1, e1), (w2, b2, a2, e2), (w3, b3, a3, e3) = lw
    if has_x:
        dx = x.shape[-1]
        w1x, w1p = w1[:dx], w1[dx:]
    else:
        w1x, w1p = None, w1
    c3 = w3.shape[-1]

    full = lambda arr: pl.BlockSpec(arr.shape, lambda b, i, j: (0,) * arr.ndim)
    in_specs = [
        pl.BlockSpec((1, 3, jsz), lambda b, i, j: (b, 0, j)),
        pl.BlockSpec((1, ti, 3), lambda b, i, j: (b, i, 0)),
    ]
    args = [pt, s]
    if has_x:
        in_specs.append(pl.BlockSpec((1, jsz, dx), lambda b, i, j: (b, j, 0)))
        args.append(x)
    in_specs += [
        pl.BlockSpec((1, ti, 1), lambda b, i, j, it=it: (b * it + i, 0, 0)),
        pl.BlockSpec((1, ti, 1), lambda b, i, j, it=it: (b * it + i, 0, 0)),
    ]
    args += [tb, jc]
    wargs = [w1p] + ([w1x] if has_x else []) + [b1, a1, e1, w2, b2, a2, e2,
                                               w3, b3, a3, e3]
    for wa in wargs:
        in_specs.append(full(wa))
    args += wargs

    return pl.pallas_call(
        functools.partial(_agg_kernel, njc=njc, ti=ti, jsz=jsz,
                          rr=np.float32(rr), has_x=has_x),
        grid=(nb, it, njc),
        in_specs=in_specs,
        out_specs=pl.BlockSpec((1, ti, c3), lambda b, i, j: (b, i, 0)),
        out_shape=jax.ShapeDtypeStruct((nb, m, c3), jnp.float32),
        interpret=interpret,
    )(*args)


def _head_kernel(x_ref, s_ref, *refs, nb, m):
    (w1x_ref, w1p_ref, b1_ref, a1_ref, e1_ref,
     w2_ref, b2_ref, a2_ref, e2_ref,
     w3_ref, b3_ref, a3_ref, e3_ref,
     l1w_ref, l1b_ref, l2w_ref, l2b_ref,
     l3w_ref, l3b_ref, l4w_ref, l4b_ref, out_ref) = refs

    x = x_ref[...]
    s = s_ref[...]
    dx = x.shape[-1]
    h = (jnp.dot(x.reshape(nb * m, dx), w1x_ref[...],
                 preferred_element_type=jnp.float32)
         + jnp.dot(s.reshape(nb * m, 3), w1p_ref[...],
                   preferred_element_type=jnp.float32) + b1_ref[...])
    h = jnp.maximum(h, 0.0) * a1_ref[...] + e1_ref[...]
    h = jnp.dot(h, w2_ref[...], preferred_element_type=jnp.float32) + b2_ref[...]
    h = jnp.maximum(h, 0.0) * a2_ref[...] + e2_ref[...]
    h = jnp.dot(h, w3_ref[...], preferred_element_type=jnp.float32) + b3_ref[...]
    h = jnp.maximum(h, 0.0) * a3_ref[...] + e3_ref[...]
    cg = h.shape[-1]
    g = jnp.max(h.reshape(nb, m, cg), axis=1)
    gc = jnp.concatenate([g[: nb // 2], g[nb // 2:]], axis=1)
    h = jnp.dot(gc, l1w_ref[...], preferred_element_type=jnp.float32) + l1b_ref[...]
    h = jnp.maximum(h, 0.0)
    h = jnp.dot(h, l2w_ref[...], preferred_element_type=jnp.float32) + l2b_ref[...]
    h = jnp.maximum(h, 0.0)
    h = jnp.dot(h, l3w_ref[...], preferred_element_type=jnp.float32) + l3b_ref[...]
    h = jnp.dot(h, l4w_ref[...], preferred_element_type=jnp.float32) + l4b_ref[...]
    mx = jnp.max(h, axis=1, keepdims=True)
    lse = jnp.log(jnp.sum(jnp.exp(h - mx), axis=1, keepdims=True)) + mx
    out_ref[...] = h - lse


def _head_call(x, s, lw, lin, interpret=False):
    nb, m, dx = x.shape
    (w1, b1, a1, e1), (w2, b2, a2, e2), (w3, b3, a3, e3) = lw
    w1x, w1p = w1[:dx], w1[dx:]
    (l1w, l1b), (l2w, l2b), (l3w, l3b), (l4w, l4b) = lin
    args = [x, s, w1x, w1p, b1, a1, e1, w2, b2, a2, e2, w3, b3, a3, e3,
            l1w, l1b, l2w, l2b, l3w, l3b, l4w, l4b]
    return pl.pallas_call(
        functools.partial(_head_kernel, nb=nb, m=m),
        out_shape=jax.ShapeDtypeStruct((nb // 2, l4w.shape[-1]), jnp.float32),
        interpret=interpret,
    )(*args)


def _prep_mlp(params):
    out = []
    c = 1.0 / np.sqrt(1.0 + EPS)
    for (w, b, g, be) in params:
        cdim = w.shape[-1]
        out.append((w, b.reshape(1, cdim), (g * c).reshape(1, cdim),
                    be.reshape(1, cdim)))
    return out


def _pointnet(pos1, pos2, params_sa1, params_sa2, params_sa3, lin_params,
              n=N, m1=M1, m2=M2, r1=R1, r2=R2, kk=KNBR, nb=2 * B,
              ti1=128, ti2=128, jsz1=128, jsz2=128, interpret=False):
    p = jnp.concatenate([pos1.reshape(nb // 2, n, 3),
                         pos2.reshape(nb // 2, n, 3)], axis=0)
    pt = jnp.transpose(p, (2, 0, 1))
    sa1 = _prep_mlp(params_sa1)
    sa2 = _prep_mlp(params_sa2)
    sa3 = _prep_mlp(params_sa3)
    lin = [(w, b.reshape(1, -1)) for (w, b) in lin_params]

    s1r = _fps_call(pt, m1, interpret)
    s1 = jnp.transpose(s1r, (1, 0, 2))
    ptb = jnp.transpose(p, (0, 2, 1))
    t1, jc1 = _sel_call(ptb, s1, r1 * r1, ti1, kk, interpret)
    x1 = _agg_call(ptb, s1, None, t1, jc1, sa1, r1 * r1, ti1, jsz1, interpret)

    s1t = jnp.transpose(s1, (2, 0, 1))
    s2r = _fps_call(s1t, m2, interpret)
    s2 = jnp.transpose(s2r, (1, 0, 2))
    s1b = jnp.transpose(s1, (0, 2, 1))
    t2, jc2 = _sel_call(s1b, s2, r2 * r2, ti2, kk, interpret)
    x2 = _agg_call(s1b, s2, x1, t2, jc2, sa2, r2 * r2, ti2, jsz2, interpret)

    return _head_call(x2, s2, sa3, lin, interpret)


def kernel(pos1, batch1, pos2, batch2, params_sa1, params_sa2, params_sa3,
           lin_params):
    return _pointnet(pos1, pos2, params_sa1, params_sa2, params_sa3,
                     lin_params)


# speedup vs baseline: 12.7761x; 1.9234x over previous
import functools

import numpy as np
import jax
import jax.numpy as jnp
from jax.experimental import pallas as pl
from jax.experimental.pallas import tpu as pltpu
from jax.experimental.pallas import tpu_sc as plsc

B, N = 8, 1024
M1, M2 = 512, 128
R1, R2 = 0.2, 0.4
KNBR = 64
EPS = 1e-5
NEG = np.float32(-np.inf)


def _fps_kernel(p_ref, s_ref, *, m, n, nb):
    p = p_ref[...]
    px, py, pz = p[0], p[1], p[2]
    iota = jax.lax.broadcasted_iota(jnp.int32, (nb, n), 1)

    def body(i, carry):
        dist, fx, fy, fz = carry
        s_ref[pl.ds(i, 1)] = jnp.concatenate([fx, fy, fz], axis=1)[None]
        d = (px - fx) ** 2 + (py - fy) ** 2 + (pz - fz) ** 2
        dist = jnp.minimum(dist, d)
        mx = jnp.max(dist, axis=1, keepdims=True)
        far = jnp.min(jnp.where(dist == mx, iota, n), axis=1, keepdims=True)
        sel = iota == far
        fx = jnp.sum(jnp.where(sel, px, 0.0), axis=1, keepdims=True)
        fy = jnp.sum(jnp.where(sel, py, 0.0), axis=1, keepdims=True)
        fz = jnp.sum(jnp.where(sel, pz, 0.0), axis=1, keepdims=True)
        return dist, fx, fy, fz

    dist0 = jnp.full((nb, n), jnp.inf, jnp.float32)
    f0 = (px[:, 0:1], py[:, 0:1], pz[:, 0:1])
    jax.lax.fori_loop(0, m, body, (dist0,) + f0)


def _fps_call(pt, m, interpret=False):
    _, nb, n = pt.shape
    return pl.pallas_call(
        functools.partial(_fps_kernel, m=m, n=n, nb=nb),
        out_shape=jax.ShapeDtypeStruct((m, nb, 3), jnp.float32),
        interpret=interpret,
    )(pt)


def _sel_kernel(p_ref, s_ref, sel_ref, *, n, ti, rr, rr_bits, kk):
    p = p_ref[0]
    s = s_ref[0]
    d2 = ((s[:, 0:1] - p[0][None, :]) ** 2
          + (s[:, 1:2] - p[1][None, :]) ** 2
          + (s[:, 2:3] - p[2][None, :]) ** 2)
    bits = jax.lax.bitcast_convert_type(d2, jnp.int32)
    inr = d2 <= rr
    iota = jax.lax.broadcasted_iota(jnp.int32, (ti, n), 1)
    counts = jnp.sum(inr.astype(jnp.int32), axis=1, keepdims=True)

    def fast():
        return (jnp.full((ti, 1), rr_bits, jnp.int32),
                jnp.full((ti, 1), n, jnp.int32))

    def slow():
        lo = jnp.zeros((ti, 1), jnp.int32)
        hi = jnp.full((ti, 1), rr_bits, jnp.int32)

        def bs(_, lh):
            lo, hi = lh
            mid = lo + (hi - lo) // 2
            c = jnp.sum((inr & (bits <= mid)).astype(jnp.int32),
                        axis=1, keepdims=True)
            ge = c >= kk
            return jnp.where(ge, lo, mid + 1), jnp.where(ge, mid, hi)

        lo, hi = jax.lax.fori_loop(0, 31, bs, (lo, hi))
        t = hi
        cnt_lt = jnp.sum((inr & (bits < t)).astype(jnp.int32),
                         axis=1, keepdims=True)
        need = kk - cnt_lt
        lo2 = jnp.zeros((ti, 1), jnp.int32)
        hi2 = jnp.full((ti, 1), n - 1, jnp.int32)

        def bs2(_, lh):
            lo2, hi2 = lh
            mid = lo2 + (hi2 - lo2) // 2
            c = jnp.sum((inr & (bits == t) & (iota <= mid)).astype(jnp.int32),
                        axis=1, keepdims=True)
            ge = c >= need
            return jnp.where(ge, lo2, mid + 1), jnp.where(ge, mid, hi2)

        lo2, hi2 = jax.lax.fori_loop(0, 11, bs2, (lo2, hi2))
        over = counts > kk
        return (jnp.where(over, t, rr_bits),
                jnp.where(over, hi2, n))

    t, jc = jax.lax.cond(jnp.max(counts) > kk, slow, fast)
    sel = inr & ((bits < t) | ((bits == t) & (iota <= jc)))
    cs = sel.astype(jnp.int32)
    lane = iota % 16
    for sft in (1, 2, 4, 8):
        sh = jnp.concatenate(
            [jnp.zeros((ti, sft), jnp.int32), cs[:, :-sft]], axis=1)
        cs = cs + jnp.where(lane >= sft, sh, 0)
    sel_ref[0] = jnp.where(sel, cs, 0)


def _sel_call(pt, s, rr, ti, kk, interpret=False):
    nb, _, n = pt.shape
    m = s.shape[1]
    it = m // ti
    rr_bits = int(np.float32(rr).view(np.int32))
    grid = (nb, it)
    return pl.pallas_call(
        functools.partial(_sel_kernel, n=n, ti=ti, rr=np.float32(rr),
                          rr_bits=rr_bits, kk=kk),
        grid=grid,
        in_specs=[
            pl.BlockSpec((1, 3, n), lambda b, i: (b, 0, 0)),
            pl.BlockSpec((1, ti, 3), lambda b, i: (b, i, 0)),
        ],
        out_specs=pl.BlockSpec((1, ti, n), lambda b, i, it=it: (b * it + i, 0, 0)),
        out_shape=jax.ShapeDtypeStruct((nb * it, ti, n), jnp.int32),
        interpret=interpret,
    )(pt, s).reshape(nb * m, n)


def _agg_kernel(*refs, njc, ti, jsz, rr, has_x):
    if has_x:
        (p_ref, s_ref, x_ref, t_ref, jc_ref, w1p_ref, w1x_ref,
         b1_ref, a1_ref, e1_ref, w2_ref, b2_ref, a2_ref, e2_ref,
         w3_ref, b3_ref, a3_ref, e3_ref, out_ref) = refs
    else:
        (p_ref, s_ref, t_ref, jc_ref, w1p_ref,
         b1_ref, a1_ref, e1_ref, w2_ref, b2_ref, a2_ref, e2_ref,
         w3_ref, b3_ref, a3_ref, e3_ref, out_ref) = refs
        x_ref = w1x_ref = None
    jcid = pl.program_id(2)

    p = p_ref[0]
    s = s_ref[0]
    d2 = ((s[:, 0:1] - p[0][None, :]) ** 2
          + (s[:, 1:2] - p[1][None, :]) ** 2
          + (s[:, 2:3] - p[2][None, :]) ** 2)
    bits = jax.lax.bitcast_convert_type(d2, jnp.int32)
    t = t_ref[0]
    jc = jc_ref[0]
    jg = jax.lax.broadcasted_iota(jnp.int32, (ti, jsz), 1) + jcid * jsz
    sel = (d2 <= rr) & ((bits < t) | ((bits == t) & (jg <= jc)))

    w1p = w1p_ref[...]
    dj = jax.lax.dot_general(p, w1p, (((0,), (0,)), ((), ())),
                             preferred_element_type=jnp.float32)
    if has_x:
        dj = dj + jnp.dot(x_ref[0], w1x_ref[...],
                          preferred_element_type=jnp.float32)
    ci = jnp.dot(s, w1p, preferred_element_type=jnp.float32)

    c1 = w1p.shape[-1]
    bc3 = lambda v, dims, sh: jax.lax.broadcast_in_dim(v, sh, dims)
    sh1 = (ti, jsz, c1)
    h = (bc3(dj, (1, 2), sh1) - bc3(ci, (0, 2), sh1)
         + bc3(b1_ref[...], (1, 2), sh1))
    h = jnp.maximum(h, 0.0) * bc3(a1_ref[...], (1, 2), sh1) \
        + bc3(e1_ref[...], (1, 2), sh1)
    h = h.reshape(ti * jsz, c1)
    h = jnp.dot(h, w2_ref[...], preferred_element_type=jnp.float32) + b2_ref[...]
    h = jnp.maximum(h, 0.0) * a2_ref[...] + e2_ref[...]
    h = jnp.dot(h, w3_ref[...], preferred_element_type=jnp.float32) + b3_ref[...]
    h = jnp.maximum(h, 0.0) * a3_ref[...] + e3_ref[...]
    c3 = h.shape[-1]
    h = h.reshape(ti, jsz, c3)
    pen = jnp.where(sel, 0.0, NEG)
    cm = jnp.max(h + jax.lax.broadcast_in_dim(pen, (ti, jsz, c3), (0, 1)),
                 axis=1)

    @pl.when(jcid == 0)
    def _():
        out_ref[0] = jnp.full((ti, c3), NEG, jnp.float32)

    out_ref[0] = jnp.maximum(out_ref[0], cm)

    @pl.when(jcid == njc - 1)
    def _():
        v = out_ref[0]
        out_ref[0] = jnp.where(v == NEG, 0.0, v)


def _agg_call(pt, s, x, tb, jc, lw, rr, ti, jsz, interpret=False):
    nb, _, n = pt.shape
    m = s.shape[1]
    it = m // ti
    njc = n // jsz
    has_x = x is not None
    (w1, b1, a1, e1), (w2, b2, a2, e2), (w3, b3, a3, e3) = lw
    if has_x:
        dx = x.shape[-1]
        w1x, w1p = w1[:dx], w1[dx:]
    else:
        w1x, w1p = None, w1
    c3 = w3.shape[-1]

    full = lambda arr: pl.BlockSpec(arr.shape, lambda b, i, j: (0,) * arr.ndim)
    in_specs = [
        pl.BlockSpec((1, 3, jsz), lambda b, i, j: (b, 0, j)),
        pl.BlockSpec((1, ti, 3), lambda b, i, j: (b, i, 0)),
    ]
    args = [pt, s]
    if has_x:
        in_specs.append(pl.BlockSpec((1, jsz, dx), lambda b, i, j: (b, j, 0)))
        args.append(x)
    in_specs += [
        pl.BlockSpec((1, ti, 1), lambda b, i, j, it=it: (b * it + i, 0, 0)),
        pl.BlockSpec((1, ti, 1), lambda b, i, j, it=it: (b * it + i, 0, 0)),
    ]
    args += [tb, jc]
    wargs = [w1p] + ([w1x] if has_x else []) + [b1, a1, e1, w2, b2, a2, e2,
                                               w3, b3, a3, e3]
    for wa in wargs:
        in_specs.append(full(wa))
    args += wargs

    return pl.pallas_call(
        functools.partial(_agg_kernel, njc=njc, ti=ti, jsz=jsz,
                          rr=np.float32(rr), has_x=has_x),
        grid=(nb, it, njc),
        in_specs=in_specs,
        out_specs=pl.BlockSpec((1, ti, c3), lambda b, i, j: (b, i, 0)),
        out_shape=jax.ShapeDtypeStruct((nb, m, c3), jnp.float32),
        interpret=interpret,
    )(*args)


def _compact_kernel(sel_ref, f_ref, g_ref, cnt_ref, selrow, ftab, nbr, gbuf,
                    cbuf, *, rpw, m, n, c, kk):
    nc = 2
    wid = jax.lax.axis_index("s") * nc + jax.lax.axis_index("c")
    base = wid * rpw
    b = base // m
    pltpu.sync_copy(f_ref.at[b], ftab)
    iota = jax.lax.iota(jnp.int32, 16)
    iotac = iota * c
    zero16 = jnp.zeros((16,), jnp.int32)
    nchunk = n // 16

    def row_body(r, _):
        row = base + r
        pltpu.sync_copy(sel_ref.at[row], selrow)
        for q in range(kk // 16 + 1):
            nbr[pl.ds(q * 16, 16)] = zero16

        def chunk(ci, off):
            pv = selrow[pl.ds(ci * 16, 16)]
            pos = jnp.where(pv > 0, off + pv - 1, kk + iota)
            plsc.store_scatter(nbr, [pos], iota + ci * 16)
            return off + jnp.max(pv)

        cnt = jax.lax.fori_loop(0, nchunk, chunk, 0)
        cbuf[...] = jax.lax.broadcast_in_dim(cnt, (16,), ())
        pltpu.sync_copy(cbuf, cnt_ref.at[row])

        for q in range(kk // 16):
            nb16c = nbr[pl.ds(q * 16, 16)] * c

            def ch_body(ch, _):
                vals = plsc.load_gather(ftab, [nb16c + ch])
                plsc.store_scatter(gbuf, [iotac + (q * 16 * c + ch)], vals)
                return 0

            jax.lax.fori_loop(0, c, ch_body, 0)
        pltpu.sync_copy(gbuf, g_ref.at[row])
        return 0

    jax.lax.fori_loop(0, rpw, row_body, 0)


def _compact_call(sel, feats, m, kk):
    R, n = sel.shape
    nb, _, c = feats.shape
    nw = 32
    rpw = R // nw
    mesh = plsc.VectorSubcoreMesh(core_axis_name="c", subcore_axis_name="s")
    kfn = pl.kernel(
        functools.partial(_compact_kernel, rpw=rpw, m=m, n=n, c=c, kk=kk),
        out_type=(jax.ShapeDtypeStruct((R, kk * c), jnp.float32),
                  jax.ShapeDtypeStruct((R, 16), jnp.int32)),
        mesh=mesh,
        compiler_params=pltpu.CompilerParams(needs_layout_passes=False),
        scratch_types=[
            pltpu.VMEM((n,), jnp.int32),
            pltpu.VMEM((n * c,), jnp.float32),
            pltpu.VMEM((kk + 16,), jnp.int32),
            pltpu.VMEM((kk * c,), jnp.float32),
            pltpu.VMEM((16,), jnp.int32),
        ],
    )
    return kfn(sel, feats.reshape(nb, n * c))


def _cagg_kernel(g_ref, cnt_ref, s_ref, w1_ref, w1p_ref, b1_ref, a1_ref,
                 e1_ref, w2_ref, b2_ref, a2_ref, e2_ref, w3_ref, b3_ref,
                 a3_ref, e3_ref, out_ref, *, ti, kk, c):
    feats = g_ref[...].reshape(ti * kk, c)
    ci = jnp.dot(s_ref[0], w1p_ref[...],
                 preferred_element_type=jnp.float32)
    h = jnp.dot(feats, w1_ref[...], preferred_element_type=jnp.float32)
    c1 = w1_ref.shape[-1]
    bc3 = lambda v, dims, sh: jax.lax.broadcast_in_dim(v, sh, dims)
    sh1 = (ti, kk, c1)
    h = (h.reshape(ti, kk, c1) - bc3(ci, (0, 2), sh1)
         + bc3(b1_ref[...], (1, 2), sh1))
    h = jnp.maximum(h, 0.0) * bc3(a1_ref[...], (1, 2), sh1) \
        + bc3(e1_ref[...], (1, 2), sh1)
    h = h.reshape(ti * kk, c1)
    h = jnp.dot(h, w2_ref[...], preferred_element_type=jnp.float32) + b2_ref[...]
    h = jnp.maximum(h, 0.0) * a2_ref[...] + e2_ref[...]
    h = jnp.dot(h, w3_ref[...], preferred_element_type=jnp.float32) + b3_ref[...]
    h = jnp.maximum(h, 0.0) * a3_ref[...] + e3_ref[...]
    c3 = w3_ref.shape[-1]
    h = h.reshape(ti, kk, c3)
    cnt1 = cnt_ref[:, 0:1]
    kio = jax.lax.broadcasted_iota(jnp.int32, (ti, kk), 1)
    pen = jnp.where(kio < cnt1, 0.0, NEG)
    mx = jnp.max(h + jax.lax.broadcast_in_dim(pen, (ti, kk, c3), (0, 1)),
                 axis=1)
    out_ref[0] = jnp.where(cnt1 > 0, mx, 0.0)


def _cagg_call(g3, cnt, s, lw, ti, kk, interpret=False):
    R, _, c = g3.shape
    nb, m, _ = s.shape
    it = m // ti
    (w1, b1, a1, e1), (w2, b2, a2, e2), (w3, b3, a3, e3) = lw
    w1p = w1[-3:]
    c3 = w3.shape[-1]
    full = lambda arr: pl.BlockSpec(arr.shape, lambda b, i: (0,) * arr.ndim)
    wargs = [w1, w1p, b1, a1, e1, w2, b2, a2, e2, w3, b3, a3, e3]
    in_specs = [
        pl.BlockSpec((ti, kk, c), lambda b, i, it=it: (b * it + i, 0, 0)),
        pl.BlockSpec((ti, 16), lambda b, i, it=it: (b * it + i, 0)),
        pl.BlockSpec((1, ti, 3), lambda b, i: (b, i, 0)),
    ] + [full(w) for w in wargs]
    return pl.pallas_call(
        functools.partial(_cagg_kernel, ti=ti, kk=kk, c=c),
        grid=(nb, it),
        in_specs=in_specs,
        out_specs=pl.BlockSpec((1, ti, c3), lambda b, i: (b, i, 0)),
        out_shape=jax.ShapeDtypeStruct((nb, m, c3), jnp.float32),
        interpret=interpret,
    )(g3, cnt, s, *wargs)


def _head_kernel(x_ref, s_ref, *refs, nb, m):
    (w1x_ref, w1p_ref, b1_ref, a1_ref, e1_ref,
     w2_ref, b2_ref, a2_ref, e2_ref,
     w3_ref, b3_ref, a3_ref, e3_ref,
     l1w_ref, l1b_ref, l2w_ref, l2b_ref,
     l3w_ref, l3b_ref, l4w_ref, l4b_ref, out_ref) = refs

    x = x_ref[...]
    s = s_ref[...]
    dx = x.shape[-1]
    h = (jnp.dot(x.reshape(nb * m, dx), w1x_ref[...],
                 preferred_element_type=jnp.float32)
         + jnp.dot(s.reshape(nb * m, 3), w1p_ref[...],
                   preferred_element_type=jnp.float32) + b1_ref[...])
    h = jnp.maximum(h, 0.0) * a1_ref[...] + e1_ref[...]
    h = jnp.dot(h, w2_ref[...], preferred_element_type=jnp.float32) + b2_ref[...]
    h = jnp.maximum(h, 0.0) * a2_ref[...] + e2_ref[...]
    h = jnp.dot(h, w3_ref[...], preferred_element_type=jnp.float32) + b3_ref[...]
    h = jnp.maximum(h, 0.0) * a3_ref[...] + e3_ref[...]
    cg = h.shape[-1]
    g = jnp.max(h.reshape(nb, m, cg), axis=1)
    gc = jnp.concatenate([g[: nb // 2], g[nb // 2:]], axis=1)
    h = jnp.dot(gc, l1w_ref[...], preferred_element_type=jnp.float32) + l1b_ref[...]
    h = jnp.maximum(h, 0.0)
    h = jnp.dot(h, l2w_ref[...], preferred_element_type=jnp.float32) + l2b_ref[...]
    h = jnp.maximum(h, 0.0)
    h = jnp.dot(h, l3w_ref[...], preferred_element_type=jnp.float32) + l3b_ref[...]
    h = jnp.dot(h, l4w_ref[...], preferred_element_type=jnp.float32) + l4b_ref[...]
    mx = jnp.max(h, axis=1, keepdims=True)
    lse = jnp.log(jnp.sum(jnp.exp(h - mx), axis=1, keepdims=True)) + mx
    out_ref[...] = h - lse


def _head_call(x, s, lw, lin, interpret=False):
    nb, m, dx = x.shape
    (w1, b1, a1, e1), (w2, b2, a2, e2), (w3, b3, a3, e3) = lw
    w1x, w1p = w1[:dx], w1[dx:]
    (l1w, l1b), (l2w, l2b), (l3w, l3b), (l4w, l4b) = lin
    args = [x, s, w1x, w1p, b1, a1, e1, w2, b2, a2, e2, w3, b3, a3, e3,
            l1w, l1b, l2w, l2b, l3w, l3b, l4w, l4b]
    return pl.pallas_call(
        functools.partial(_head_kernel, nb=nb, m=m),
        out_shape=jax.ShapeDtypeStruct((nb // 2, l4w.shape[-1]), jnp.float32),
        interpret=interpret,
    )(*args)


def _prep_mlp(params):
    out = []
    c = 1.0 / np.sqrt(1.0 + EPS)
    for (w, b, g, be) in params:
        cdim = w.shape[-1]
        out.append((w, b.reshape(1, cdim), (g * c).reshape(1, cdim),
                    be.reshape(1, cdim)))
    return out


def _pointnet(pos1, pos2, params_sa1, params_sa2, params_sa3, lin_params,
              n=N, m1=M1, m2=M2, r1=R1, r2=R2, kk=KNBR, nb=2 * B,
              ti1=128, ti2=128, jsz1=128, jsz2=128, interpret=False):
    p = jnp.concatenate([pos1.reshape(nb // 2, n, 3),
                         pos2.reshape(nb // 2, n, 3)], axis=0)
    pt = jnp.transpose(p, (2, 0, 1))
    sa1 = _prep_mlp(params_sa1)
    sa2 = _prep_mlp(params_sa2)
    sa3 = _prep_mlp(params_sa3)
    lin = [(w, b.reshape(1, -1)) for (w, b) in lin_params]

    s1r = _fps_call(pt, m1, interpret)
    s1 = jnp.transpose(s1r, (1, 0, 2))
    ptb = jnp.transpose(p, (0, 2, 1))
    sel1 = _sel_call(ptb, s1, r1 * r1, ti1, kk, interpret)
    g1, cnt1 = _compact_call(sel1, p, m1, kk)
    x1 = _cagg_call(g1.reshape(nb * m1, kk, 3), cnt1, s1, sa1, ti1, kk,
                    interpret)

    s1t = jnp.transpose(s1, (2, 0, 1))
    s2r = _fps_call(s1t, m2, interpret)
    s2 = jnp.transpose(s2r, (1, 0, 2))
    s1b = jnp.transpose(s1, (0, 2, 1))
    sel2 = _sel_call(s1b, s2, r2 * r2, ti2, kk, interpret)
    f2 = jnp.concatenate([x1, s1], axis=-1)
    g2, cnt2 = _compact_call(sel2, f2, m2, kk)
    x2 = _cagg_call(g2.reshape(nb * m2, kk, f2.shape[-1]), cnt2, s2, sa2,
                    ti2, kk, interpret)

    return _head_call(x2, s2, sa3, lin, interpret)


def kernel(pos1, batch1, pos2, batch2, params_sa1, params_sa2, params_sa3,
           lin_params):
    return _pointnet(pos1, pos2, params_sa1, params_sa2, params_sa3,
                     lin_params)


# speedup vs baseline: 16.3193x; 1.2773x over previous
import functools

import numpy as np
import jax
import jax.numpy as jnp
from jax.experimental import pallas as pl
from jax.experimental.pallas import tpu as pltpu
from jax.experimental.pallas import tpu_sc as plsc

B, N = 8, 1024
M1, M2 = 512, 128
R1, R2 = 0.2, 0.4
KNBR = 64
EPS = 1e-5
NEG = np.float32(-np.inf)


def _fps_kernel(p_ref, s_ref, *, m, n, nb):
    p = p_ref[...]
    px, py, pz = p[0], p[1], p[2]
    iota = jax.lax.broadcasted_iota(jnp.int32, (nb, n), 1)

    def body(i, carry):
        dist, fx, fy, fz = carry
        s_ref[pl.ds(i, 1)] = jnp.concatenate([fx, fy, fz], axis=1)[None]
        d = (px - fx) ** 2 + (py - fy) ** 2 + (pz - fz) ** 2
        dist = jnp.minimum(dist, d)
        mx = jnp.max(dist, axis=1, keepdims=True)
        far = jnp.min(jnp.where(dist == mx, iota, n), axis=1, keepdims=True)
        sel = iota == far
        fx = jnp.sum(jnp.where(sel, px, 0.0), axis=1, keepdims=True)
        fy = jnp.sum(jnp.where(sel, py, 0.0), axis=1, keepdims=True)
        fz = jnp.sum(jnp.where(sel, pz, 0.0), axis=1, keepdims=True)
        return dist, fx, fy, fz

    dist0 = jnp.full((nb, n), jnp.inf, jnp.float32)
    f0 = (px[:, 0:1], py[:, 0:1], pz[:, 0:1])
    jax.lax.fori_loop(0, m, body, (dist0,) + f0)


def _fps_call(pt, m, interpret=False):
    _, nb, n = pt.shape
    return pl.pallas_call(
        functools.partial(_fps_kernel, m=m, n=n, nb=nb),
        out_shape=jax.ShapeDtypeStruct((m, nb, 3), jnp.float32),
        interpret=interpret,
    )(pt)


def _sel_kernel(p_ref, s_ref, sel_ref, *, n, ti, rr, rr_bits, kk):
    p = p_ref[0]
    s = s_ref[0]
    d2 = ((s[:, 0:1] - p[0][None, :]) ** 2
          + (s[:, 1:2] - p[1][None, :]) ** 2
          + (s[:, 2:3] - p[2][None, :]) ** 2)
    bits = jax.lax.bitcast_convert_type(d2, jnp.int32)
    inr = d2 <= rr
    iota = jax.lax.broadcasted_iota(jnp.int32, (ti, n), 1)
    counts = jnp.sum(inr.astype(jnp.int32), axis=1, keepdims=True)

    def fast():
        return (jnp.full((ti, 1), rr_bits, jnp.int32),
                jnp.full((ti, 1), n, jnp.int32))

    def slow():
        lo = jnp.zeros((ti, 1), jnp.int32)
        hi = jnp.full((ti, 1), rr_bits, jnp.int32)

        def bs(_, lh):
            lo, hi = lh
            mid = lo + (hi - lo) // 2
            c = jnp.sum((inr & (bits <= mid)).astype(jnp.int32),
                        axis=1, keepdims=True)
            ge = c >= kk
            return jnp.where(ge, lo, mid + 1), jnp.where(ge, mid, hi)

        lo, hi = jax.lax.fori_loop(0, 31, bs, (lo, hi))
        t = hi
        cnt_lt = jnp.sum((inr & (bits < t)).astype(jnp.int32),
                         axis=1, keepdims=True)
        need = kk - cnt_lt
        lo2 = jnp.zeros((ti, 1), jnp.int32)
        hi2 = jnp.full((ti, 1), n - 1, jnp.int32)

        def bs2(_, lh):
            lo2, hi2 = lh
            mid = lo2 + (hi2 - lo2) // 2
            c = jnp.sum((inr & (bits == t) & (iota <= mid)).astype(jnp.int32),
                        axis=1, keepdims=True)
            ge = c >= need
            return jnp.where(ge, lo2, mid + 1), jnp.where(ge, mid, hi2)

        lo2, hi2 = jax.lax.fori_loop(0, 11, bs2, (lo2, hi2))
        over = counts > kk
        return (jnp.where(over, t, rr_bits),
                jnp.where(over, hi2, n))

    t, jc = jax.lax.cond(jnp.max(counts) > kk, slow, fast)
    sel = inr & ((bits < t) | ((bits == t) & (iota <= jc)))
    cs = sel.astype(jnp.int32)
    lane = iota % 16
    for sft in (1, 2, 4, 8):
        sh = jnp.concatenate(
            [jnp.zeros((ti, sft), jnp.int32), cs[:, :-sft]], axis=1)
        cs = cs + jnp.where(lane >= sft, sh, 0)
    sel_ref[0] = jnp.where(sel, cs, 0)


def _sel_call(pt, s, rr, ti, kk, interpret=False):
    nb, _, n = pt.shape
    m = s.shape[1]
    it = m // ti
    rr_bits = int(np.float32(rr).view(np.int32))
    grid = (nb, it)
    return pl.pallas_call(
        functools.partial(_sel_kernel, n=n, ti=ti, rr=np.float32(rr),
                          rr_bits=rr_bits, kk=kk),
        grid=grid,
        in_specs=[
            pl.BlockSpec((1, 3, n), lambda b, i: (b, 0, 0)),
            pl.BlockSpec((1, ti, 3), lambda b, i: (b, i, 0)),
        ],
        out_specs=pl.BlockSpec((1, ti, n), lambda b, i, it=it: (b * it + i, 0, 0)),
        out_shape=jax.ShapeDtypeStruct((nb * it, ti, n), jnp.int32),
        interpret=interpret,
    )(pt, s).reshape(nb * m, n)


def _compact_kernel(*refs, rpw, m, n, cx, cp, kk):
    if cx:
        (sel_ref, fx_ref, fp_ref, gx_ref, gp_ref, cnt_ref,
         selrow0, selrow1, ftab, nbr0, nbr1, gx0, gx1, gp0, gp1,
         cbuf0, cbuf1, sin0, sin1, sgx0, sgx1, sox0, sox1, sop0, sop1,
         scn0, scn1) = refs
    else:
        (sel_ref, fp_ref, gp_ref, cnt_ref,
         selrow0, selrow1, ftab, nbr0, nbr1, gp0, gp1,
         cbuf0, cbuf1, sin0, sin1, sop0, sop1, scn0, scn1) = refs
        fx_ref = gx_ref = gx0 = gx1 = None
        sgx0 = sgx1 = sox0 = sox1 = None
    nc = 2
    wid = jax.lax.axis_index("s") * nc + jax.lax.axis_index("c")
    base = wid * rpw
    b = base // m
    jbase = b * n if cx else 0
    pltpu.sync_copy(fp_ref.at[b], ftab)
    iota = jax.lax.iota(jnp.int32, 16)
    iotac = iota * cp
    pad16 = jnp.full((16,), jbase, jnp.int32)
    nchunk4 = n // 64

    def scan(selrow, nbr):
        for q in range(kk // 16 + 1):
            nbr[pl.ds(q * 16, 16)] = pad16

        def chunk4(c4, off):
            for qq in range(4):
                ci = c4 * 4 + qq
                pv = selrow[pl.ds(ci * 16, 16)]
                pos = jnp.where(pv > 0, off + pv - 1, kk + iota)
                plsc.store_scatter(nbr, [pos], iota + (ci * 16 + jbase))
                off = off + jnp.max(pv)
            return off

        return jax.lax.fori_loop(0, nchunk4, chunk4, 0)

    def gather_pos(nbr, gp):
        for q in range(kk // 16):
            nb16c = (nbr[pl.ds(q * 16, 16)] - jbase) * cp
            for ch in range(cp):
                vals = plsc.load_gather(ftab, [nb16c + ch])
                plsc.store_scatter(gp, [iotac + (q * 16 * cp + ch)], vals)

    def in_cp(r, selrow, sem):
        return pltpu.make_async_copy(sel_ref.at[base + r], selrow, sem)

    def gx_cp(nbr, gx, sem):
        return pltpu.make_async_copy(fx_ref.at[nbr.at[pl.ds(0, kk)]], gx, sem)

    def outx_cp(r, gx, sem):
        return pltpu.make_async_copy(gx, gx_ref.at[base + r], sem)

    def outp_cp(r, gp, sem):
        return pltpu.make_async_copy(gp, gp_ref.at[base + r], sem)

    def cnt_cp(r, cbuf, sem):
        return pltpu.make_async_copy(cbuf, cnt_ref.at[base + r], sem)

    in_cp(0, selrow0, sin0).start()
    nit = rpw // 2
    bufs = ((selrow0, nbr0, gx0, gp0, cbuf0, sin0, sgx0, sox0, sop0, scn0),
            (selrow1, nbr1, gx1, gp1, cbuf1, sin1, sgx1, sox1, sop1, scn1))

    def row_pair(i, _):
        r0 = i * 2
        for q, (selrow, nbr, gx, gp, cbuf, sin, sgx, sox, sop, scn) in \
                enumerate(bufs):
            r = r0 + q
            in_cp(r, selrow, sin).wait()
            if q == 0:
                in_cp(r + 1, selrow1, sin1).start()
            else:
                @pl.when(i + 1 < nit)
                def _():
                    in_cp(r + 1, selrow0, sin0).start()

            cnt = scan(selrow, nbr)
            if cx:
                @pl.when(i > 0)
                def _():
                    outx_cp(r - 2, gx, sox).wait()
                gx_cp(nbr, gx, sgx).start()

            @pl.when(i > 0)
            def _():
                outp_cp(r - 2, gp, sop).wait()

            gather_pos(nbr, gp)
            outp_cp(r, gp, sop).start()

            @pl.when(i > 0)
            def _():
                cnt_cp(r - 2, cbuf, scn).wait()

            cbuf[...] = jax.lax.broadcast_in_dim(cnt, (16,), ())
            cnt_cp(r, cbuf, scn).start()
        if cx:
            for q, (selrow, nbr, gx, gp, cbuf, sin, sgx, sox, sop, scn) in \
                    enumerate(bufs):
                gx_cp(nbr, gx, sgx).wait()
                outx_cp(r0 + q, gx, sox).start()
        return 0

    jax.lax.fori_loop(0, nit, row_pair, 0)
    outp_cp(rpw - 2, gp0, sop0).wait()
    outp_cp(rpw - 1, gp1, sop1).wait()
    if cx:
        outx_cp(rpw - 2, gx0, sox0).wait()
        outx_cp(rpw - 1, gx1, sox1).wait()
    cnt_cp(rpw - 2, cbuf0, scn0).wait()
    cnt_cp(rpw - 1, cbuf1, scn1).wait()


def _compact_call(sel, fx, fp, m, kk):
    R, n = sel.shape
    nb, _, cp = fp.shape
    cx = 0 if fx is None else fx.shape[-1]
    rpw = R // 32
    mesh = plsc.VectorSubcoreMesh(core_axis_name="c", subcore_axis_name="s")
    outs = []
    if cx:
        outs.append(jax.ShapeDtypeStruct((R, kk, cx), jnp.float32))
    outs += [jax.ShapeDtypeStruct((R, kk * cp), jnp.float32),
             jax.ShapeDtypeStruct((R, 16), jnp.int32)]
    scr = ([pltpu.VMEM((n,), jnp.int32)] * 2
           + [pltpu.VMEM((n * cp,), jnp.float32)]
           + [pltpu.VMEM((kk + 16,), jnp.int32)] * 2)
    if cx:
        scr += [pltpu.VMEM((kk, cx), jnp.float32)] * 2
    scr += ([pltpu.VMEM((kk * cp,), jnp.float32)] * 2
            + [pltpu.VMEM((16,), jnp.int32)] * 2
            + [pltpu.SemaphoreType.DMA] * (10 if cx else 6))
    kfn = pl.kernel(
        functools.partial(_compact_kernel, rpw=rpw, m=m, n=n, cx=cx, cp=cp,
                          kk=kk),
        out_type=tuple(outs),
        mesh=mesh,
        compiler_params=pltpu.CompilerParams(needs_layout_passes=False),
        scratch_types=scr,
    )
    args = (sel,)
    if cx:
        args += (fx.reshape(nb * n, cx),)
    args += (fp.reshape(nb, n * cp),)
    return kfn(*args)


def _cagg_kernel(*refs, ti, kk, cx, cp):
    if cx:
        (gx_ref, gp_ref, cnt_ref, s_ref, w1x_ref, w1p_ref, b1_ref, a1_ref,
         e1_ref, w2_ref, b2_ref, a2_ref, e2_ref, w3_ref, b3_ref, a3_ref,
         e3_ref, out_ref) = refs
    else:
        (gp_ref, cnt_ref, s_ref, w1p_ref, b1_ref, a1_ref,
         e1_ref, w2_ref, b2_ref, a2_ref, e2_ref, w3_ref, b3_ref, a3_ref,
         e3_ref, out_ref) = refs
        gx_ref = w1x_ref = None
    w1p = w1p_ref[...]
    ci = jnp.dot(s_ref[0], w1p, preferred_element_type=jnp.float32)
    h = jnp.dot(gp_ref[...].reshape(ti * kk, cp), w1p,
                preferred_element_type=jnp.float32)
    if cx:
        h = h + jnp.dot(gx_ref[...].reshape(ti * kk, cx), w1x_ref[...],
                        preferred_element_type=jnp.float32)
    c1 = w1p.shape[-1]
    bc3 = lambda v, dims, sh: jax.lax.broadcast_in_dim(v, sh, dims)
    sh1 = (ti, kk, c1)
    h = (h.reshape(ti, kk, c1) - bc3(ci, (0, 2), sh1)
         + bc3(b1_ref[...], (1, 2), sh1))
    h = jnp.maximum(h, 0.0) * bc3(a1_ref[...], (1, 2), sh1) \
        + bc3(e1_ref[...], (1, 2), sh1)
    h = h.reshape(ti * kk, c1)
    h = jnp.dot(h, w2_ref[...], preferred_element_type=jnp.float32) + b2_ref[...]
    h = jnp.maximum(h, 0.0) * a2_ref[...] + e2_ref[...]
    h = jnp.dot(h, w3_ref[...], preferred_element_type=jnp.float32) + b3_ref[...]
    h = jnp.maximum(h, 0.0) * a3_ref[...] + e3_ref[...]
    c3 = w3_ref.shape[-1]
    h = h.reshape(ti, kk, c3)
    cnt1 = cnt_ref[:, 0:1]
    kio = jax.lax.broadcasted_iota(jnp.int32, (ti, kk), 1)
    pen = jnp.where(kio < cnt1, 0.0, NEG)
    mx = jnp.max(h + jax.lax.broadcast_in_dim(pen, (ti, kk, c3), (0, 1)),
                 axis=1)
    out_ref[0] = jnp.where(cnt1 > 0, mx, 0.0)


def _cagg_call(gx, gp3, cnt, s, lw, ti, kk, interpret=False):
    R = gp3.shape[0]
    cp = gp3.shape[-1]
    cx = 0 if gx is None else gx.shape[-1]
    nb, m, _ = s.shape
    it = m // ti
    (w1, b1, a1, e1), (w2, b2, a2, e2), (w3, b3, a3, e3) = lw
    w1x, w1p = (w1[:cx], w1[cx:]) if cx else (None, w1)
    c3 = w3.shape[-1]
    full = lambda arr: pl.BlockSpec(arr.shape, lambda b, i: (0,) * arr.ndim)
    in_specs = []
    args = []
    if cx:
        in_specs.append(
            pl.BlockSpec((ti, kk, cx), lambda b, i, it=it: (b * it + i, 0, 0)))
        args.append(gx)
    in_specs += [
        pl.BlockSpec((ti, kk, cp), lambda b, i, it=it: (b * it + i, 0, 0)),
        pl.BlockSpec((ti, 16), lambda b, i, it=it: (b * it + i, 0)),
        pl.BlockSpec((1, ti, 3), lambda b, i: (b, i, 0)),
    ]
    args += [gp3, cnt, s]
    wargs = ([w1x] if cx else []) + [w1p, b1, a1, e1, w2, b2, a2, e2,
                                     w3, b3, a3, e3]
    in_specs += [full(w) for w in wargs]
    args += wargs
    return pl.pallas_call(
        functools.partial(_cagg_kernel, ti=ti, kk=kk, cx=cx, cp=cp),
        grid=(nb, it),
        in_specs=in_specs,
        out_specs=pl.BlockSpec((1, ti, c3), lambda b, i: (b, i, 0)),
        out_shape=jax.ShapeDtypeStruct((nb, m, c3), jnp.float32),
        interpret=interpret,
    )(*args)


def _head_kernel(x_ref, s_ref, *refs, nb, m):
    (w1x_ref, w1p_ref, b1_ref, a1_ref, e1_ref,
     w2_ref, b2_ref, a2_ref, e2_ref,
     w3_ref, b3_ref, a3_ref, e3_ref,
     l1w_ref, l1b_ref, l2w_ref, l2b_ref,
     l3w_ref, l3b_ref, l4w_ref, l4b_ref, out_ref) = refs

    x = x_ref[...]
    s = s_ref[...]
    dx = x.shape[-1]
    h = (jnp.dot(x.reshape(nb * m, dx), w1x_ref[...],
                 preferred_element_type=jnp.float32)
         + jnp.dot(s.reshape(nb * m, 3), w1p_ref[...],
                   preferred_element_type=jnp.float32) + b1_ref[...])
    h = jnp.maximum(h, 0.0) * a1_ref[...] + e1_ref[...]
    h = jnp.dot(h, w2_ref[...], preferred_element_type=jnp.float32) + b2_ref[...]
    h = jnp.maximum(h, 0.0) * a2_ref[...] + e2_ref[...]
    h = jnp.dot(h, w3_ref[...], preferred_element_type=jnp.float32) + b3_ref[...]
    h = jnp.maximum(h, 0.0) * a3_ref[...] + e3_ref[...]
    cg = h.shape[-1]
    g = jnp.max(h.reshape(nb, m, cg), axis=1)
    gc = jnp.concatenate([g[: nb // 2], g[nb // 2:]], axis=1)
    h = jnp.dot(gc, l1w_ref[...], preferred_element_type=jnp.float32) + l1b_ref[...]
    h = jnp.maximum(h, 0.0)
    h = jnp.dot(h, l2w_ref[...], preferred_element_type=jnp.float32) + l2b_ref[...]
    h = jnp.maximum(h, 0.0)
    h = jnp.dot(h, l3w_ref[...], preferred_element_type=jnp.float32) + l3b_ref[...]
    h = jnp.dot(h, l4w_ref[...], preferred_element_type=jnp.float32) + l4b_ref[...]
    mx = jnp.max(h, axis=1, keepdims=True)
    lse = jnp.log(jnp.sum(jnp.exp(h - mx), axis=1, keepdims=True)) + mx
    out_ref[...] = h - lse


def _head_call(x, s, lw, lin, interpret=False):
    nb, m, dx = x.shape
    (w1, b1, a1, e1), (w2, b2, a2, e2), (w3, b3, a3, e3) = lw
    w1x, w1p = w1[:dx], w1[dx:]
    (l1w, l1b), (l2w, l2b), (l3w, l3b), (l4w, l4b) = lin
    args = [x, s, w1x, w1p, b1, a1, e1, w2, b2, a2, e2, w3, b3, a3, e3,
            l1w, l1b, l2w, l2b, l3w, l3b, l4w, l4b]
    return pl.pallas_call(
        functools.partial(_head_kernel, nb=nb, m=m),
        out_shape=jax.ShapeDtypeStruct((nb // 2, l4w.shape[-1]), jnp.float32),
        interpret=interpret,
    )(*args)


def _prep_mlp(params):
    out = []
    c = 1.0 / np.sqrt(1.0 + EPS)
    for (w, b, g, be) in params:
        cdim = w.shape[-1]
        out.append((w, b.reshape(1, cdim), (g * c).reshape(1, cdim),
                    be.reshape(1, cdim)))
    return out


def _pointnet(pos1, pos2, params_sa1, params_sa2, params_sa3, lin_params,
              n=N, m1=M1, m2=M2, r1=R1, r2=R2, kk=KNBR, nb=2 * B,
              ti1=128, ti2=128, jsz1=128, jsz2=128, interpret=False):
    p = jnp.concatenate([pos1.reshape(nb // 2, n, 3),
                         pos2.reshape(nb // 2, n, 3)], axis=0)
    pt = jnp.transpose(p, (2, 0, 1))
    sa1 = _prep_mlp(params_sa1)
    sa2 = _prep_mlp(params_sa2)
    sa3 = _prep_mlp(params_sa3)
    lin = [(w, b.reshape(1, -1)) for (w, b) in lin_params]

    s1r = _fps_call(pt, m1, interpret)
    s1 = jnp.transpose(s1r, (1, 0, 2))
    ptb = jnp.transpose(p, (0, 2, 1))
    sel1 = _sel_call(ptb, s1, r1 * r1, ti1, kk, interpret)
    gp1, cnt1 = _compact_call(sel1, None, p, m1, kk)
    x1 = _cagg_call(None, gp1.reshape(nb * m1, kk, 3), cnt1, s1, sa1, ti1,
                    kk, interpret)

    s1t = jnp.transpose(s1, (2, 0, 1))
    s2r = _fps_call(s1t, m2, interpret)
    s2 = jnp.transpose(s2r, (1, 0, 2))
    s1b = jnp.transpose(s1, (0, 2, 1))
    sel2 = _sel_call(s1b, s2, r2 * r2, ti2, kk, interpret)
    gx2, gp2, cnt2 = _compact_call(sel2, x1, s1, m2, kk)
    x2 = _cagg_call(gx2, gp2.reshape(nb * m2, kk, 3), cnt2, s2, sa2,
                    ti2, kk, interpret)

    return _head_call(x2, s2, sa3, lin, interpret)


def kernel(pos1, batch1, pos2, batch2, params_sa1, params_sa2, params_sa3,
           lin_params):
    return _pointnet(pos1, pos2, params_sa1, params_sa2, params_sa3,
                     lin_params)
